# baseline (speedup 1.0000x reference)
import jax
import jax.numpy as jnp
from jax import lax
from jax.experimental import pallas as pl
from jax.experimental.pallas import tpu as pltpu
from jax.experimental.pallas import tpu_sc as plsc

ENT = 1_000_000
BATCH = 16384
DIM = 64
_INFO = plsc.get_sparse_core_info()
NC, NS, NL = _INFO.num_cores, _INFO.num_subcores, _INFO.num_lanes
NW = NC * NS
B_PER_W = BATCH // NW
IDX_CHUNK = 128
N_CHUNKS = B_PER_W // IDX_CHUNK

COLSUM_BLOCK = 32768
SC_BLOCKS = 13
SC_ENT = SC_BLOCKS * COLSUM_BLOCK
TC_ENT = ENT - SC_ENT
W_PER_W = SC_ENT // NW
CS_CHUNK = 512
N_CS_CHUNKS = W_PER_W // CS_CHUNK
NBUF = 2


def _colsum_tc_body(x_ref, o_ref):
    o_ref[...] = -jnp.sum(x_ref[...], axis=0)


def _colsum_sc_body(table_hbm, out_hbm, buf_v, acc_v, sem0, sem1):
    wid = lax.axis_index("s") * NC + lax.axis_index("c")
    col0 = wid * W_PER_W
    sems = (sem0, sem1)

    def chunk_copy(c, b):
        return pltpu.make_async_copy(
            table_hbm.at[:, pl.ds(col0 + c * CS_CHUNK, CS_CHUNK)],
            buf_v.at[b],
            sems[b])

    for b in range(NBUF):
        chunk_copy(b, b).start()

    def outer(i, carry):
        for b in range(NBUF):
            c = i * NBUF + b
            chunk_copy(c, b).wait()

            bb = buf_v.at[b]

            def group(g, carry2):
                sl = pl.ds(g * NL, NL)
                a = [bb[d, sl] for d in range(4)]
                for d in range(4, DIM, 4):
                    for k in range(4):
                        a[k] = a[k] + bb[d + k, sl]
                acc = (a[0] + a[1]) + (a[2] + a[3])
                acc_v[pl.ds(c * CS_CHUNK + g * NL, NL)] = -acc
                return carry2

            lax.fori_loop(0, CS_CHUNK // NL, group, 0, unroll=False)

            @pl.when(c + NBUF < N_CS_CHUNKS)
            def _():
                chunk_copy(c + NBUF, b).start()
        return carry

    lax.fori_loop(0, N_CS_CHUNKS // NBUF, outer, 0, unroll=False)

    pltpu.sync_copy(acc_v, out_hbm.at[pl.ds(col0, W_PER_W)])


def _gather_body(cs_sc_hbm, cs_tc_hbm, idx_hbm, out_hbm,
                 idx_v, idx_a, idx_b, vals_a, vals_b, sem):
    wid = lax.axis_index("s") * NC + lax.axis_index("c")
    base = wid * B_PER_W

    pltpu.sync_copy(idx_hbm.at[pl.ds(base, B_PER_W)], idx_v)

    for j in range(N_CHUNKS):
        def clamp_body(g, carry, j=j):
            h = idx_v[pl.ds(j * IDX_CHUNK + g * NL, NL)]
            a1 = jnp.where(h >= SC_ENT, h - SC_ENT, h)
            idx_b.at[j][pl.ds(g * NL, NL)] = a1
            idx_a.at[j][pl.ds(g * NL, NL)] = jnp.where(
                a1 >= SC_ENT, a1 - SC_ENT, a1)
            return carry

        lax.fori_loop(0, IDX_CHUNK // NL, clamp_body, 0, unroll=False)

    copies = []
    for j in range(N_CHUNKS):
        sl = pl.ds(j * IDX_CHUNK, IDX_CHUNK)
        copies.append(pltpu.async_copy(
            cs_sc_hbm.at[idx_a.at[j]], vals_a.at[sl], sem))
        copies.append(pltpu.async_copy(
            cs_tc_hbm.at[idx_b.at[j]], vals_b.at[sl], sem))
    for c in copies:
        c.wait()

    def sel_body(g, carry):
        sl = pl.ds(g * NL, NL)
        h = idx_v[sl]
        vals_a[sl] = jnp.where(h < SC_ENT, vals_a[sl], vals_b[sl])
        return carry

    lax.fori_loop(0, B_PER_W // NL, sel_body, 0, unroll=False)

    pltpu.sync_copy(vals_a, out_hbm.at[pl.ds(base, B_PER_W)])


@jax.jit
def _score(ent_embeddings, batch_h):
    table_t = ent_embeddings.T

    mesh = plsc.VectorSubcoreMesh(core_axis_name="c", subcore_axis_name="s")

    colsum_tc = pl.pallas_call(
        _colsum_tc_body,
        out_shape=jax.ShapeDtypeStruct((TC_ENT,), jnp.float32),
        grid=(pl.cdiv(TC_ENT, COLSUM_BLOCK),),
        in_specs=[pl.BlockSpec((DIM, COLSUM_BLOCK),
                               lambda i: (0, i + SC_BLOCKS))],
        out_specs=pl.BlockSpec((COLSUM_BLOCK,), lambda i: (i,)),
    )(table_t)

    colsum_sc = pl.kernel(
        _colsum_sc_body,
        out_type=jax.ShapeDtypeStruct((SC_ENT,), jnp.float32),
        mesh=mesh,
        scratch_types=[
            pltpu.VMEM((NBUF, DIM, CS_CHUNK), jnp.float32),
            pltpu.VMEM((W_PER_W,), jnp.float32),
            pltpu.SemaphoreType.DMA,
            pltpu.SemaphoreType.DMA,
        ],
    )(table_t)

    run = pl.kernel(
        _gather_body,
        out_type=jax.ShapeDtypeStruct((BATCH,), jnp.float32),
        mesh=mesh,
        scratch_types=[
            pltpu.VMEM((B_PER_W,), jnp.int32),
            pltpu.VMEM((N_CHUNKS, IDX_CHUNK), jnp.int32),
            pltpu.VMEM((N_CHUNKS, IDX_CHUNK), jnp.int32),
            pltpu.VMEM((B_PER_W,), jnp.float32),
            pltpu.VMEM((B_PER_W,), jnp.float32),
            pltpu.SemaphoreType.DMA,
        ],
    )
    return run(colsum_sc, colsum_tc, batch_h)


def kernel(ent_embeddings, rel_matrices, batch_h, batch_t, batch_r):
    return _score(ent_embeddings, batch_h)


# speedup vs baseline: 1.0462x; 1.0462x over previous
import jax
import jax.numpy as jnp
from jax import lax
from jax.experimental import pallas as pl
from jax.experimental.pallas import tpu as pltpu
from jax.experimental.pallas import tpu_sc as plsc

ENT = 1_000_000
BATCH = 16384
DIM = 64
_INFO = plsc.get_sparse_core_info()
NC, NS, NL = _INFO.num_cores, _INFO.num_subcores, _INFO.num_lanes
NW = NC * NS
B_PER_W = BATCH // NW
IDX_CHUNK = 128
N_CHUNKS = B_PER_W // IDX_CHUNK

COLSUM_BLOCK = 32768


def _colsum_body(x_ref, o_ref):
    o_ref[...] = -jnp.sum(x_ref[...], axis=0)


def _gather_body(colsum_hbm, idx_hbm, out_hbm, idx_v, vals_v, sem):
    wid = lax.axis_index("s") * NC + lax.axis_index("c")
    base = wid * B_PER_W

    for j in range(N_CHUNKS):
        pltpu.sync_copy(idx_hbm.at[pl.ds(base + j * IDX_CHUNK, IDX_CHUNK)],
                        idx_v.at[j])

    copies = []
    for j in range(N_CHUNKS):
        copies.append(pltpu.async_copy(
            colsum_hbm.at[idx_v.at[j]],
            vals_v.at[pl.ds(j * IDX_CHUNK, IDX_CHUNK)],
            sem))
    for c in copies:
        c.wait()

    pltpu.sync_copy(vals_v, out_hbm.at[pl.ds(base, B_PER_W)])


@jax.jit
def _score(ent_embeddings, batch_h):
    table_t = ent_embeddings.T

    colsum = pl.pallas_call(
        _colsum_body,
        out_shape=jax.ShapeDtypeStruct((ENT,), jnp.float32),
        grid=(pl.cdiv(ENT, COLSUM_BLOCK),),
        in_specs=[pl.BlockSpec((DIM, COLSUM_BLOCK), lambda i: (0, i))],
        out_specs=pl.BlockSpec((COLSUM_BLOCK,), lambda i: (i,)),
    )(table_t)

    mesh = plsc.VectorSubcoreMesh(core_axis_name="c", subcore_axis_name="s")
    run = pl.kernel(
        _gather_body,
        out_type=jax.ShapeDtypeStruct((BATCH,), jnp.float32),
        mesh=mesh,
        scratch_types=[
            pltpu.VMEM((N_CHUNKS, IDX_CHUNK), jnp.int32),
            pltpu.VMEM((B_PER_W,), jnp.float32),
            pltpu.SemaphoreType.DMA,
        ],
    )
    return run(colsum, batch_h)


def kernel(ent_embeddings, rel_matrices, batch_h, batch_t, batch_r):
    return _score(ent_embeddings, batch_h)
